# baseline (speedup 1.0000x reference)
import functools

import jax
import jax.numpy as jnp
from jax.experimental import pallas as pl
from jax.experimental.pallas import tpu as pltpu

F = 128
NHEADS = 3
SCW = NHEADS * F + 8


def _embed_body(fea_ref, w_ref, b_ref, wt_ref, out_ref):
    out = jnp.dot(fea_ref[...], w_ref[...], preferred_element_type=jnp.float32)
    out = out + b_ref[...]
    col = jax.lax.broadcasted_iota(jnp.int32, out.shape, 1)
    out_ref[...] = jnp.where(col == F - 1, wt_ref[...], out)


def _embed(elem_fea, elem_weights, emb_W, emb_b):
    n = elem_fea.shape[0]
    bn = 1000
    w_pad = jnp.pad(emb_W, ((0, 0), (0, 1)))
    b_pad = jnp.pad(emb_b, (0, 1)).reshape(1, F)
    return pl.pallas_call(
        _embed_body,
        grid=(n // bn,),
        in_specs=[
            pl.BlockSpec((bn, F), lambda i: (i, 0)),
            pl.BlockSpec((F, F), lambda i: (0, 0)),
            pl.BlockSpec((1, F), lambda i: (0, 0)),
            pl.BlockSpec((bn, 1), lambda i: (i, 0)),
        ],
        out_specs=pl.BlockSpec((bn, F), lambda i: (i, 0)),
        out_shape=jax.ShapeDtypeStruct((n, F), jnp.float32),
        compiler_params=pltpu.CompilerParams(
            dimension_semantics=("parallel",)),
    )(elem_fea, w_pad, b_pad, elem_weights)


def _heads_body(fea_ref, wg1_ref, bg1_ref, w2c_ref, wm1_ref, bm1_ref,
                wm2_ref, bm2_ref, gates_ref, msgs_ref, gmax_ref, *, cdtype):
    fea = fea_ref[...].astype(cdtype)
    hgs, msgs = [], []
    for h in range(NHEADS):
        hg = jnp.dot(fea, wg1_ref[h].astype(cdtype),
                     preferred_element_type=jnp.float32) + bg1_ref[h]
        hg = hg * jax.nn.sigmoid(hg)
        hgs.append(hg.astype(cdtype))
        hm = jnp.dot(fea, wm1_ref[h].astype(cdtype),
                     preferred_element_type=jnp.float32) + bm1_ref[h]
        hm = hm * jax.nn.sigmoid(hm)
        m = jnp.dot(hm.astype(cdtype), wm2_ref[h].astype(cdtype),
                    preferred_element_type=jnp.float32) + bm2_ref[h]
        msgs.append(m)
    hg_cat = jnp.concatenate(hgs, axis=1)
    gates = jnp.dot(hg_cat, w2c_ref[...].astype(cdtype),
                    preferred_element_type=jnp.float32)
    gates_ref[...] = gates
    msgs_ref[...] = jnp.concatenate(msgs, axis=1).astype(msgs_ref.dtype)
    bmax = jnp.max(gates, axis=0, keepdims=True)

    @pl.when(pl.program_id(0) == 0)
    def _():
        gmax_ref[...] = bmax

    @pl.when(pl.program_id(0) > 0)
    def _():
        gmax_ref[...] = jnp.maximum(gmax_ref[...], bmax)


def _stack_heads(heads):
    wg1 = jnp.stack([h["gate"]["hidden"][0][0] for h in heads])
    bg1 = jnp.stack([h["gate"]["hidden"][0][1] for h in heads])
    hid = wg1.shape[-1]
    w2c = jnp.concatenate([
        jnp.pad(h["gate"]["out"][0], ((0, 0), (i, 8 - 1 - i)))
        for i, h in enumerate(heads)], axis=0)
    wm1 = jnp.stack([h["msg"]["hidden"][0][0] for h in heads])
    bm1 = jnp.stack([h["msg"]["hidden"][0][1] for h in heads])
    wm2 = jnp.stack([h["msg"]["out"][0] for h in heads])
    bm2 = jnp.stack([h["msg"]["out"][1] for h in heads])
    return (wg1, bg1, w2c, wm1, bm1, wm2, bm2), hid


def _heads_forward(fea, stacked, hid, be, cdtype):
    e, din = fea.shape
    wg1, bg1, w2c, wm1, bm1, wm2, bm2 = stacked
    full = lambda *shape: pl.BlockSpec(shape, lambda i: (0,) * len(shape))
    return pl.pallas_call(
        functools.partial(_heads_body, cdtype=cdtype),
        grid=(e // be,),
        in_specs=[
            pl.BlockSpec((be, din), lambda i: (i, 0)),
            full(NHEADS, din, hid), full(NHEADS, hid),
            full(NHEADS * hid, 8),
            full(NHEADS, din, hid), full(NHEADS, hid),
            full(NHEADS, hid, F), full(NHEADS, F),
        ],
        out_specs=[
            pl.BlockSpec((be, 8), lambda i: (i, 0)),
            pl.BlockSpec((be, NHEADS * F), lambda i: (i, 0)),
            pl.BlockSpec((1, 8), lambda i: (0, 0)),
        ],
        out_shape=[
            jax.ShapeDtypeStruct((e, 8), jnp.float32),
            jax.ShapeDtypeStruct((e, NHEADS * F), jnp.bfloat16),
            jax.ShapeDtypeStruct((1, 8), jnp.float32),
        ],
        compiler_params=pltpu.CompilerParams(
            dimension_semantics=("arbitrary",)),
    )(fea, wg1, bg1, w2c, wm1, bm1, wm2, bm2)


def _sscan_body(gates_ref, gmax_ref, w_ref, msgs_ref, idx_ref, out_ref,
                carry_ref, cidx_ref, *, sub, nsub):
    @pl.when(pl.program_id(0) == 0)
    def _():
        carry_ref[...] = jnp.zeros_like(carry_ref)
        cidx_ref[...] = jnp.full_like(cidx_ref, -1)

    p8 = w_ref[...] * jnp.exp(gates_ref[...] - gmax_ref[...])
    m = msgs_ref[...].astype(jnp.float32)
    parts = [p8[:, h:h + 1] * m[:, h * F:(h + 1) * F] for h in range(NHEADS)]
    parts.append(p8)
    val = jnp.concatenate(parts, axis=1)
    idx = idx_ref[...]
    ri = jax.lax.broadcasted_iota(jnp.int32, (sub, sub), 0)
    ci = jax.lax.broadcasted_iota(jnp.int32, (sub, sub), 1)
    tri = ci <= ri
    for g in range(nsub):
        idxg = idx[g * sub:(g + 1) * sub, :]
        valg = val[g * sub:(g + 1) * sub, :].astype(jnp.bfloat16)
        idr = jax.lax.broadcast_in_dim(idxg[:, 0], (sub, sub), (0,))
        idc = jax.lax.broadcast_in_dim(idxg[:, 0], (sub, sub), (1,))
        mask = jnp.logical_and(idr == idc, tri).astype(jnp.bfloat16)
        ssum = jnp.dot(mask, valg, preferred_element_type=jnp.float32)
        ssum = ssum + ((idxg == cidx_ref[...]).astype(jnp.float32)
                       * carry_ref[...])
        out_ref[g * sub:(g + 1) * sub, :] = ssum
        carry_ref[...] = ssum[sub - 1:sub, :]
        cidx_ref[...] = idxg[sub - 1:sub, :]


def _seg_scan(gates8, gmax, edge_w, msgs, idx2, be, sub):
    e = gates8.shape[0]
    return pl.pallas_call(
        functools.partial(_sscan_body, sub=sub, nsub=be // sub),
        grid=(e // be,),
        in_specs=[
            pl.BlockSpec((be, 8), lambda i: (i, 0)),
            pl.BlockSpec((1, 8), lambda i: (0, 0)),
            pl.BlockSpec((be, 1), lambda i: (i, 0)),
            pl.BlockSpec((be, NHEADS * F), lambda i: (i, 0)),
            pl.BlockSpec((be, 1), lambda i: (i, 0)),
        ],
        out_specs=pl.BlockSpec((be, SCW), lambda i: (i, 0)),
        out_shape=jax.ShapeDtypeStruct((e, SCW), jnp.float32),
        scratch_shapes=[
            pltpu.VMEM((1, SCW), jnp.float32),
            pltpu.VMEM((1, 1), jnp.int32),
        ],
        compiler_params=pltpu.CompilerParams(
            dimension_semantics=("arbitrary",)),
    )(gates8, gmax, edge_w, msgs, idx2)


def _wap(fea, idx, ends, counts, edge_w, stacked, hid, be, sub, cdtype):
    gates8, msgs, gmax = _heads_forward(fea, stacked, hid, be, cdtype)
    idx2 = idx.reshape(-1, 1)
    ssum = _seg_scan(gates8, gmax, edge_w, msgs, idx2, be, sub)
    pos = jnp.clip(ends - 1, 0, ssum.shape[0] - 1)
    s = jnp.where((counts > 0)[:, None], ssum[pos], 0.0)
    acc = s[:, 0:F] / (s[:, NHEADS * F:NHEADS * F + 1] + 1e-10)
    for h in range(1, NHEADS):
        acc = acc + (s[:, h * F:(h + 1) * F]
                     / (s[:, NHEADS * F + h:NHEADS * F + h + 1] + 1e-10))
    return acc * (1.0 / NHEADS)


def kernel(elem_weights, elem_fea, self_fea_idx, nbr_fea_idx, cry_elem_idx,
           params):
    cdtype = jnp.bfloat16
    n = elem_fea.shape[0]
    emb_W, emb_b = params["emb"]
    x = _embed(elem_fea, elem_weights, emb_W, emb_b)

    nbr_w = elem_weights[nbr_fea_idx]
    e = self_fea_idx.shape[0]
    counts = jnp.zeros((n,), jnp.int32).at[self_fea_idx].add(1)
    ends = jnp.cumsum(counts)
    for heads in params["graphs"]:
        stacked, hid = _stack_heads(heads)
        fea = jnp.concatenate([x[self_fea_idx], x[nbr_fea_idx]], axis=1)
        pooled = _wap(fea, self_fea_idx, ends, counts, nbr_w, stacked, hid,
                      be=1600, sub=320, cdtype=cdtype)
        x = pooled + x

    cry_counts = jnp.zeros((2000,), jnp.int32).at[cry_elem_idx].add(1)
    cry_ends = jnp.cumsum(cry_counts)
    cry_stacked, cry_hid = _stack_heads(params["cry"])
    cry_fea = _wap(x, cry_elem_idx, cry_ends, cry_counts, elem_weights,
                   cry_stacked, cry_hid, be=1000, sub=250, cdtype=cdtype)
    return (cry_fea, x)


# speedup vs baseline: 1.2545x; 1.2545x over previous
import functools

import jax
import jax.numpy as jnp
from jax import lax
from jax.experimental import pallas as pl
from jax.experimental.pallas import tpu as pltpu
from jax.experimental.pallas import tpu_sc as plsc

F = 128
NHEADS = 3
SCW = NHEADS * F + 8


def _embed_body(fea_ref, w_ref, b_ref, wt_ref, out_ref):
    out = jnp.dot(fea_ref[...], w_ref[...], preferred_element_type=jnp.float32)
    out = out + b_ref[...]
    col = jax.lax.broadcasted_iota(jnp.int32, out.shape, 1)
    out_ref[...] = jnp.where(col == F - 1, wt_ref[...], out)


def _embed(elem_fea, elem_weights, emb_W, emb_b):
    n = elem_fea.shape[0]
    bn = 1000
    w_pad = jnp.pad(emb_W, ((0, 0), (0, 1)))
    b_pad = jnp.pad(emb_b, (0, 1)).reshape(1, F)
    return pl.pallas_call(
        _embed_body,
        grid=(n // bn,),
        in_specs=[
            pl.BlockSpec((bn, F), lambda i: (i, 0)),
            pl.BlockSpec((F, F), lambda i: (0, 0)),
            pl.BlockSpec((1, F), lambda i: (0, 0)),
            pl.BlockSpec((bn, 1), lambda i: (i, 0)),
        ],
        out_specs=pl.BlockSpec((bn, F), lambda i: (i, 0)),
        out_shape=jax.ShapeDtypeStruct((n, F), jnp.float32),
        compiler_params=pltpu.CompilerParams(
            dimension_semantics=("parallel",)),
    )(elem_fea, w_pad, b_pad, elem_weights)


_GC = 400


def _sc_gather(table, idx_il):
    rows = idx_il.shape[0]
    info = plsc.get_sparse_core_info()
    nw = info.num_cores * info.num_subcores
    per_w = rows // nw
    nchunk = per_w // _GC
    mesh = plsc.VectorSubcoreMesh(core_axis_name="c", subcore_axis_name="s")

    @functools.partial(
        pl.kernel, mesh=mesh,
        out_type=jax.ShapeDtypeStruct((rows, F), jnp.float32),
        scratch_types=[
            pltpu.VMEM((_GC,), jnp.int32),
            pltpu.VMEM((_GC, F), jnp.float32),
            pltpu.SemaphoreType.DMA,
        ],
    )
    def gk(table_hbm, idx_hbm, out_hbm, idx_v, rows_v, sem):
        wid = lax.axis_index("s") * info.num_cores + lax.axis_index("c")
        base0 = wid * per_w

        def body(c, carry):
            base = base0 + c * _GC
            pltpu.sync_copy(idx_hbm.at[pl.ds(base, _GC)], idx_v)
            pltpu.async_copy(table_hbm.at[idx_v], rows_v, sem).wait()
            pltpu.sync_copy(rows_v, out_hbm.at[pl.ds(base, _GC)])
            return carry

        lax.fori_loop(0, nchunk, body, 0)

    return gk(table, idx_il)


def _heads_body(fea_ref, wg1_ref, bg1_ref, w2c_ref, wm1_ref, bm1_ref,
                wm2_ref, bm2_ref, gates_ref, msgs_ref, gmax_ref, *, cdtype):
    fea = fea_ref[...].astype(cdtype)
    hgs, msgs = [], []
    for h in range(NHEADS):
        hg = jnp.dot(fea, wg1_ref[h].astype(cdtype),
                     preferred_element_type=jnp.float32) + bg1_ref[h]
        hg = hg * jax.nn.sigmoid(hg)
        hgs.append(hg.astype(cdtype))
        hm = jnp.dot(fea, wm1_ref[h].astype(cdtype),
                     preferred_element_type=jnp.float32) + bm1_ref[h]
        hm = hm * jax.nn.sigmoid(hm)
        m = jnp.dot(hm.astype(cdtype), wm2_ref[h].astype(cdtype),
                    preferred_element_type=jnp.float32) + bm2_ref[h]
        msgs.append(m)
    hg_cat = jnp.concatenate(hgs, axis=1)
    gates = jnp.dot(hg_cat, w2c_ref[...].astype(cdtype),
                    preferred_element_type=jnp.float32)
    gates_ref[...] = gates
    msgs_ref[...] = jnp.concatenate(msgs, axis=1).astype(msgs_ref.dtype)
    bmax = jnp.max(gates, axis=0, keepdims=True)

    @pl.when(pl.program_id(0) == 0)
    def _():
        gmax_ref[...] = bmax

    @pl.when(pl.program_id(0) > 0)
    def _():
        gmax_ref[...] = jnp.maximum(gmax_ref[...], bmax)


def _stack_heads(heads):
    wg1 = jnp.stack([h["gate"]["hidden"][0][0] for h in heads])
    bg1 = jnp.stack([h["gate"]["hidden"][0][1] for h in heads])
    hid = wg1.shape[-1]
    w2c = jnp.concatenate([
        jnp.pad(h["gate"]["out"][0], ((0, 0), (i, 8 - 1 - i)))
        for i, h in enumerate(heads)], axis=0)
    wm1 = jnp.stack([h["msg"]["hidden"][0][0] for h in heads])
    bm1 = jnp.stack([h["msg"]["hidden"][0][1] for h in heads])
    wm2 = jnp.stack([h["msg"]["out"][0] for h in heads])
    bm2 = jnp.stack([h["msg"]["out"][1] for h in heads])
    return (wg1, bg1, w2c, wm1, bm1, wm2, bm2), hid


def _heads_forward(fea, stacked, hid, be, cdtype):
    e, din = fea.shape
    wg1, bg1, w2c, wm1, bm1, wm2, bm2 = stacked
    full = lambda *shape: pl.BlockSpec(shape, lambda i: (0,) * len(shape))
    return pl.pallas_call(
        functools.partial(_heads_body, cdtype=cdtype),
        grid=(e // be,),
        in_specs=[
            pl.BlockSpec((be, din), lambda i: (i, 0)),
            full(NHEADS, din, hid), full(NHEADS, hid),
            full(NHEADS * hid, 8),
            full(NHEADS, din, hid), full(NHEADS, hid),
            full(NHEADS, hid, F), full(NHEADS, F),
        ],
        out_specs=[
            pl.BlockSpec((be, 8), lambda i: (i, 0)),
            pl.BlockSpec((be, NHEADS * F), lambda i: (i, 0)),
            pl.BlockSpec((1, 8), lambda i: (0, 0)),
        ],
        out_shape=[
            jax.ShapeDtypeStruct((e, 8), jnp.float32),
            jax.ShapeDtypeStruct((e, NHEADS * F), jnp.bfloat16),
            jax.ShapeDtypeStruct((1, 8), jnp.float32),
        ],
        compiler_params=pltpu.CompilerParams(
            dimension_semantics=("arbitrary",)),
    )(fea, wg1, bg1, w2c, wm1, bm1, wm2, bm2)


def _sscan_body(gates_ref, gmax_ref, w_ref, msgs_ref, idx_ref, out_ref,
                carry_ref, cidx_ref, *, sub, nsub):
    @pl.when(pl.program_id(0) == 0)
    def _():
        carry_ref[...] = jnp.zeros_like(carry_ref)
        cidx_ref[...] = jnp.full_like(cidx_ref, -1)

    p8 = w_ref[...] * jnp.exp(gates_ref[...] - gmax_ref[...])
    m = msgs_ref[...].astype(jnp.float32)
    parts = [p8[:, h:h + 1] * m[:, h * F:(h + 1) * F] for h in range(NHEADS)]
    parts.append(p8)
    val = jnp.concatenate(parts, axis=1)
    idx = idx_ref[...]
    ri = jax.lax.broadcasted_iota(jnp.int32, (sub, sub), 0)
    ci = jax.lax.broadcasted_iota(jnp.int32, (sub, sub), 1)
    tri = ci <= ri
    for g in range(nsub):
        idxg = idx[g * sub:(g + 1) * sub, :]
        valg = val[g * sub:(g + 1) * sub, :].astype(jnp.bfloat16)
        idr = jax.lax.broadcast_in_dim(idxg[:, 0], (sub, sub), (0,))
        idc = jax.lax.broadcast_in_dim(idxg[:, 0], (sub, sub), (1,))
        mask = jnp.logical_and(idr == idc, tri).astype(jnp.bfloat16)
        ssum = jnp.dot(mask, valg, preferred_element_type=jnp.float32)
        ssum = ssum + ((idxg == cidx_ref[...]).astype(jnp.float32)
                       * carry_ref[...])
        out_ref[g * sub:(g + 1) * sub, :] = ssum
        carry_ref[...] = ssum[sub - 1:sub, :]
        cidx_ref[...] = idxg[sub - 1:sub, :]


def _seg_scan(gates8, gmax, edge_w, msgs, idx2, be, sub):
    e = gates8.shape[0]
    return pl.pallas_call(
        functools.partial(_sscan_body, sub=sub, nsub=be // sub),
        grid=(e // be,),
        in_specs=[
            pl.BlockSpec((be, 8), lambda i: (i, 0)),
            pl.BlockSpec((1, 8), lambda i: (0, 0)),
            pl.BlockSpec((be, 1), lambda i: (i, 0)),
            pl.BlockSpec((be, NHEADS * F), lambda i: (i, 0)),
            pl.BlockSpec((be, 1), lambda i: (i, 0)),
        ],
        out_specs=pl.BlockSpec((be, SCW), lambda i: (i, 0)),
        out_shape=jax.ShapeDtypeStruct((e, SCW), jnp.float32),
        scratch_shapes=[
            pltpu.VMEM((1, SCW), jnp.float32),
            pltpu.VMEM((1, 1), jnp.int32),
        ],
        compiler_params=pltpu.CompilerParams(
            dimension_semantics=("arbitrary",)),
    )(gates8, gmax, edge_w, msgs, idx2)


def _wap(fea, idx, ends, counts, edge_w, stacked, hid, be, sub, cdtype):
    gates8, msgs, gmax = _heads_forward(fea, stacked, hid, be, cdtype)
    idx2 = idx.reshape(-1, 1)
    ssum = _seg_scan(gates8, gmax, edge_w, msgs, idx2, be, sub)
    pos = jnp.clip(ends - 1, 0, ssum.shape[0] - 1)
    s = jnp.where((counts > 0)[:, None], ssum[pos], 0.0)
    acc = s[:, 0:F] / (s[:, NHEADS * F:NHEADS * F + 1] + 1e-10)
    for h in range(1, NHEADS):
        acc = acc + (s[:, h * F:(h + 1) * F]
                     / (s[:, NHEADS * F + h:NHEADS * F + h + 1] + 1e-10))
    return acc * (1.0 / NHEADS)


def kernel(elem_weights, elem_fea, self_fea_idx, nbr_fea_idx, cry_elem_idx,
           params):
    cdtype = jnp.bfloat16
    n = elem_fea.shape[0]
    emb_W, emb_b = params["emb"]
    x = _embed(elem_fea, elem_weights, emb_W, emb_b)

    nbr_w = elem_weights[nbr_fea_idx]
    e = self_fea_idx.shape[0]
    idx_il = jnp.stack([self_fea_idx, nbr_fea_idx], axis=1).reshape(-1)
    counts = jnp.zeros((n,), jnp.int32).at[self_fea_idx].add(1)
    ends = jnp.cumsum(counts)
    for heads in params["graphs"]:
        stacked, hid = _stack_heads(heads)
        fea = _sc_gather(x, idx_il).reshape(e, 2 * F)
        pooled = _wap(fea, self_fea_idx, ends, counts, nbr_w, stacked, hid,
                      be=1600, sub=320, cdtype=cdtype)
        x = pooled + x

    cry_counts = jnp.zeros((2000,), jnp.int32).at[cry_elem_idx].add(1)
    cry_ends = jnp.cumsum(cry_counts)
    cry_stacked, cry_hid = _stack_heads(params["cry"])
    cry_fea = _wap(x, cry_elem_idx, cry_ends, cry_counts, elem_weights,
                   cry_stacked, cry_hid, be=1000, sub=250, cdtype=cdtype)
    return (cry_fea, x)


# speedup vs baseline: 1.2588x; 1.0034x over previous
import functools

import jax
import jax.numpy as jnp
from jax import lax
from jax.experimental import pallas as pl
from jax.experimental.pallas import tpu as pltpu
from jax.experimental.pallas import tpu_sc as plsc

F = 128
NHEADS = 3
SCW = NHEADS * F + 8


def _embed_body(fea_ref, w_ref, b_ref, wt_ref, out_ref):
    out = jnp.dot(fea_ref[...], w_ref[...], preferred_element_type=jnp.float32)
    out = out + b_ref[...]
    col = jax.lax.broadcasted_iota(jnp.int32, out.shape, 1)
    out_ref[...] = jnp.where(col == F - 1, wt_ref[...], out)


def _embed(elem_fea, elem_weights, emb_W, emb_b):
    n = elem_fea.shape[0]
    bn = 1000
    w_pad = jnp.pad(emb_W, ((0, 0), (0, 1)))
    b_pad = jnp.pad(emb_b, (0, 1)).reshape(1, F)
    return pl.pallas_call(
        _embed_body,
        grid=(n // bn,),
        in_specs=[
            pl.BlockSpec((bn, F), lambda i: (i, 0)),
            pl.BlockSpec((F, F), lambda i: (0, 0)),
            pl.BlockSpec((1, F), lambda i: (0, 0)),
            pl.BlockSpec((bn, 1), lambda i: (i, 0)),
        ],
        out_specs=pl.BlockSpec((bn, F), lambda i: (i, 0)),
        out_shape=jax.ShapeDtypeStruct((n, F), jnp.float32),
        compiler_params=pltpu.CompilerParams(
            dimension_semantics=("parallel",)),
    )(elem_fea, w_pad, b_pad, elem_weights)


_GC = 200


def _sc_gather(table, idx_il):
    rows = idx_il.shape[0]
    info = plsc.get_sparse_core_info()
    nw = info.num_cores * info.num_subcores
    per_w = rows // nw
    npair = per_w // (2 * _GC)
    mesh = plsc.VectorSubcoreMesh(core_axis_name="c", subcore_axis_name="s")

    @functools.partial(
        pl.kernel, mesh=mesh,
        out_type=jax.ShapeDtypeStruct((rows, F), jnp.float32),
        scratch_types=[
            pltpu.VMEM((_GC,), jnp.int32),
            pltpu.VMEM((_GC,), jnp.int32),
            pltpu.VMEM((_GC, F), jnp.float32),
            pltpu.VMEM((_GC, F), jnp.float32),
            pltpu.SemaphoreType.DMA,
            pltpu.SemaphoreType.DMA,
        ],
    )
    def gk(table_hbm, idx_hbm, out_hbm, idx_a, idx_b, rows_a, rows_b,
           sem_a, sem_b):
        wid = lax.axis_index("s") * info.num_cores + lax.axis_index("c")
        base0 = wid * per_w

        def body(c2, carry):
            b0 = base0 + c2 * 2 * _GC
            b1 = b0 + _GC
            pltpu.sync_copy(idx_hbm.at[pl.ds(b0, _GC)], idx_a)
            cp0 = pltpu.async_copy(table_hbm.at[idx_a], rows_a, sem_a)
            pltpu.sync_copy(idx_hbm.at[pl.ds(b1, _GC)], idx_b)
            cp1 = pltpu.async_copy(table_hbm.at[idx_b], rows_b, sem_b)
            cp0.wait()
            pltpu.sync_copy(rows_a, out_hbm.at[pl.ds(b0, _GC)])
            cp1.wait()
            pltpu.sync_copy(rows_b, out_hbm.at[pl.ds(b1, _GC)])
            return carry

        lax.fori_loop(0, npair, body, 0)

    return gk(table, idx_il)


def _heads_body(fea_ref, wg1_ref, bg1_ref, w2c_ref, wm1_ref, bm1_ref,
                wm2_ref, bm2_ref, gates_ref, msgs_ref, gmax_ref, *, cdtype):
    fea = fea_ref[...].astype(cdtype)
    hgs, msgs = [], []
    for h in range(NHEADS):
        hg = jnp.dot(fea, wg1_ref[h].astype(cdtype),
                     preferred_element_type=jnp.float32) + bg1_ref[h]
        hg = hg * jax.nn.sigmoid(hg)
        hgs.append(hg.astype(cdtype))
        hm = jnp.dot(fea, wm1_ref[h].astype(cdtype),
                     preferred_element_type=jnp.float32) + bm1_ref[h]
        hm = hm * jax.nn.sigmoid(hm)
        m = jnp.dot(hm.astype(cdtype), wm2_ref[h].astype(cdtype),
                    preferred_element_type=jnp.float32) + bm2_ref[h]
        msgs.append(m)
    hg_cat = jnp.concatenate(hgs, axis=1)
    gates = jnp.dot(hg_cat, w2c_ref[...].astype(cdtype),
                    preferred_element_type=jnp.float32)
    gates_ref[...] = gates
    msgs_ref[...] = jnp.concatenate(msgs, axis=1).astype(msgs_ref.dtype)
    bmax = jnp.max(gates, axis=0, keepdims=True)

    @pl.when(pl.program_id(0) == 0)
    def _():
        gmax_ref[...] = bmax

    @pl.when(pl.program_id(0) > 0)
    def _():
        gmax_ref[...] = jnp.maximum(gmax_ref[...], bmax)


def _stack_heads(heads):
    wg1 = jnp.stack([h["gate"]["hidden"][0][0] for h in heads])
    bg1 = jnp.stack([h["gate"]["hidden"][0][1] for h in heads])
    hid = wg1.shape[-1]
    w2c = jnp.concatenate([
        jnp.pad(h["gate"]["out"][0], ((0, 0), (i, 8 - 1 - i)))
        for i, h in enumerate(heads)], axis=0)
    wm1 = jnp.stack([h["msg"]["hidden"][0][0] for h in heads])
    bm1 = jnp.stack([h["msg"]["hidden"][0][1] for h in heads])
    wm2 = jnp.stack([h["msg"]["out"][0] for h in heads])
    bm2 = jnp.stack([h["msg"]["out"][1] for h in heads])
    return (wg1, bg1, w2c, wm1, bm1, wm2, bm2), hid


def _heads_forward(fea, stacked, hid, be, cdtype):
    e, din = fea.shape
    wg1, bg1, w2c, wm1, bm1, wm2, bm2 = stacked
    full = lambda *shape: pl.BlockSpec(shape, lambda i: (0,) * len(shape))
    return pl.pallas_call(
        functools.partial(_heads_body, cdtype=cdtype),
        grid=(e // be,),
        in_specs=[
            pl.BlockSpec((be, din), lambda i: (i, 0)),
            full(NHEADS, din, hid), full(NHEADS, hid),
            full(NHEADS * hid, 8),
            full(NHEADS, din, hid), full(NHEADS, hid),
            full(NHEADS, hid, F), full(NHEADS, F),
        ],
        out_specs=[
            pl.BlockSpec((be, 8), lambda i: (i, 0)),
            pl.BlockSpec((be, NHEADS * F), lambda i: (i, 0)),
            pl.BlockSpec((1, 8), lambda i: (0, 0)),
        ],
        out_shape=[
            jax.ShapeDtypeStruct((e, 8), jnp.float32),
            jax.ShapeDtypeStruct((e, NHEADS * F), jnp.bfloat16),
            jax.ShapeDtypeStruct((1, 8), jnp.float32),
        ],
        compiler_params=pltpu.CompilerParams(
            dimension_semantics=("arbitrary",)),
    )(fea, wg1, bg1, w2c, wm1, bm1, wm2, bm2)


def _sscan_body(gates_ref, gmax_ref, w_ref, msgs_ref, idx_ref, out_ref,
                carry_ref, cidx_ref, *, sub, nsub):
    @pl.when(pl.program_id(0) == 0)
    def _():
        carry_ref[...] = jnp.zeros_like(carry_ref)
        cidx_ref[...] = jnp.full_like(cidx_ref, -1)

    p8 = w_ref[...] * jnp.exp(gates_ref[...] - gmax_ref[...])
    m = msgs_ref[...].astype(jnp.float32)
    parts = [p8[:, h:h + 1] * m[:, h * F:(h + 1) * F] for h in range(NHEADS)]
    parts.append(p8)
    val = jnp.concatenate(parts, axis=1)
    idx = idx_ref[...]
    ri = jax.lax.broadcasted_iota(jnp.int32, (sub, sub), 0)
    ci = jax.lax.broadcasted_iota(jnp.int32, (sub, sub), 1)
    tri = ci <= ri
    for g in range(nsub):
        idxg = idx[g * sub:(g + 1) * sub, :]
        valg = val[g * sub:(g + 1) * sub, :].astype(jnp.bfloat16)
        idr = jax.lax.broadcast_in_dim(idxg[:, 0], (sub, sub), (0,))
        idc = jax.lax.broadcast_in_dim(idxg[:, 0], (sub, sub), (1,))
        mask = jnp.logical_and(idr == idc, tri).astype(jnp.bfloat16)
        ssum = jnp.dot(mask, valg, preferred_element_type=jnp.float32)
        ssum = ssum + ((idxg == cidx_ref[...]).astype(jnp.float32)
                       * carry_ref[...])
        out_ref[g * sub:(g + 1) * sub, :] = ssum
        carry_ref[...] = ssum[sub - 1:sub, :]
        cidx_ref[...] = idxg[sub - 1:sub, :]


def _seg_scan(gates8, gmax, edge_w, msgs, idx2, be, sub):
    e = gates8.shape[0]
    return pl.pallas_call(
        functools.partial(_sscan_body, sub=sub, nsub=be // sub),
        grid=(e // be,),
        in_specs=[
            pl.BlockSpec((be, 8), lambda i: (i, 0)),
            pl.BlockSpec((1, 8), lambda i: (0, 0)),
            pl.BlockSpec((be, 1), lambda i: (i, 0)),
            pl.BlockSpec((be, NHEADS * F), lambda i: (i, 0)),
            pl.BlockSpec((be, 1), lambda i: (i, 0)),
        ],
        out_specs=pl.BlockSpec((be, SCW), lambda i: (i, 0)),
        out_shape=jax.ShapeDtypeStruct((e, SCW), jnp.float32),
        scratch_shapes=[
            pltpu.VMEM((1, SCW), jnp.float32),
            pltpu.VMEM((1, 1), jnp.int32),
        ],
        compiler_params=pltpu.CompilerParams(
            dimension_semantics=("arbitrary",)),
    )(gates8, gmax, edge_w, msgs, idx2)


def _wap(fea, idx, ends, counts, edge_w, stacked, hid, be, sub, cdtype):
    gates8, msgs, gmax = _heads_forward(fea, stacked, hid, be, cdtype)
    idx2 = idx.reshape(-1, 1)
    ssum = _seg_scan(gates8, gmax, edge_w, msgs, idx2, be, sub)
    pos = jnp.clip(ends - 1, 0, ssum.shape[0] - 1)
    s = jnp.where((counts > 0)[:, None], ssum[pos], 0.0)
    acc = s[:, 0:F] / (s[:, NHEADS * F:NHEADS * F + 1] + 1e-10)
    for h in range(1, NHEADS):
        acc = acc + (s[:, h * F:(h + 1) * F]
                     / (s[:, NHEADS * F + h:NHEADS * F + h + 1] + 1e-10))
    return acc * (1.0 / NHEADS)


def kernel(elem_weights, elem_fea, self_fea_idx, nbr_fea_idx, cry_elem_idx,
           params):
    cdtype = jnp.bfloat16
    n = elem_fea.shape[0]
    emb_W, emb_b = params["emb"]
    x = _embed(elem_fea, elem_weights, emb_W, emb_b)

    nbr_w = elem_weights[nbr_fea_idx]
    e = self_fea_idx.shape[0]
    idx_il = jnp.stack([self_fea_idx, nbr_fea_idx], axis=1).reshape(-1)
    counts = jnp.zeros((n,), jnp.int32).at[self_fea_idx].add(1)
    ends = jnp.cumsum(counts)
    for heads in params["graphs"]:
        stacked, hid = _stack_heads(heads)
        fea = _sc_gather(x, idx_il).reshape(e, 2 * F)
        pooled = _wap(fea, self_fea_idx, ends, counts, nbr_w, stacked, hid,
                      be=1600, sub=320, cdtype=cdtype)
        x = pooled + x

    cry_counts = jnp.zeros((2000,), jnp.int32).at[cry_elem_idx].add(1)
    cry_ends = jnp.cumsum(cry_counts)
    cry_stacked, cry_hid = _stack_heads(params["cry"])
    cry_fea = _wap(x, cry_elem_idx, cry_ends, cry_counts, elem_weights,
                   cry_stacked, cry_hid, be=1000, sub=250, cdtype=cdtype)
    return (cry_fea, x)


# speedup vs baseline: 1.3124x; 1.0426x over previous
import functools

import jax
import jax.numpy as jnp
from jax import lax
from jax.experimental import pallas as pl
from jax.experimental.pallas import tpu as pltpu
from jax.experimental.pallas import tpu_sc as plsc

F = 128
NHEADS = 3
SCW = NHEADS * F + 8


def _embed_body(fea_ref, w_ref, b_ref, wt_ref, out_ref):
    out = jnp.dot(fea_ref[...], w_ref[...], preferred_element_type=jnp.float32)
    out = out + b_ref[...]
    col = jax.lax.broadcasted_iota(jnp.int32, out.shape, 1)
    out_ref[...] = jnp.where(col == F - 1, wt_ref[...], out)


def _embed(elem_fea, elem_weights, emb_W, emb_b):
    n = elem_fea.shape[0]
    bn = 1000
    w_pad = jnp.pad(emb_W, ((0, 0), (0, 1)))
    b_pad = jnp.pad(emb_b, (0, 1)).reshape(1, F)
    return pl.pallas_call(
        _embed_body,
        grid=(n // bn,),
        in_specs=[
            pl.BlockSpec((bn, F), lambda i: (i, 0)),
            pl.BlockSpec((F, F), lambda i: (0, 0)),
            pl.BlockSpec((1, F), lambda i: (0, 0)),
            pl.BlockSpec((bn, 1), lambda i: (i, 0)),
        ],
        out_specs=pl.BlockSpec((bn, F), lambda i: (i, 0)),
        out_shape=jax.ShapeDtypeStruct((n, F), jnp.float32),
        compiler_params=pltpu.CompilerParams(
            dimension_semantics=("parallel",)),
    )(elem_fea, w_pad, b_pad, elem_weights)


_GC = 200


def _sc_gather(table, idx_il):
    rows = idx_il.shape[0]
    info = plsc.get_sparse_core_info()
    nw = info.num_cores * info.num_subcores
    per_w = rows // nw
    npair = per_w // (2 * _GC)
    mesh = plsc.VectorSubcoreMesh(core_axis_name="c", subcore_axis_name="s")

    @functools.partial(
        pl.kernel, mesh=mesh,
        out_type=jax.ShapeDtypeStruct((rows, F), jnp.float32),
        scratch_types=[
            pltpu.VMEM((_GC,), jnp.int32),
            pltpu.VMEM((_GC,), jnp.int32),
            pltpu.VMEM((_GC, F), jnp.float32),
            pltpu.VMEM((_GC, F), jnp.float32),
            pltpu.SemaphoreType.DMA,
            pltpu.SemaphoreType.DMA,
        ],
    )
    def gk(table_hbm, idx_hbm, out_hbm, idx_a, idx_b, rows_a, rows_b,
           sem_a, sem_b):
        wid = lax.axis_index("s") * info.num_cores + lax.axis_index("c")
        base0 = wid * per_w

        def body(c2, carry):
            b0 = base0 + c2 * 2 * _GC
            b1 = b0 + _GC
            pltpu.sync_copy(idx_hbm.at[pl.ds(b0, _GC)], idx_a)
            cp0 = pltpu.async_copy(table_hbm.at[idx_a], rows_a, sem_a)
            pltpu.sync_copy(idx_hbm.at[pl.ds(b1, _GC)], idx_b)
            cp1 = pltpu.async_copy(table_hbm.at[idx_b], rows_b, sem_b)
            cp0.wait()
            pltpu.sync_copy(rows_a, out_hbm.at[pl.ds(b0, _GC)])
            cp1.wait()
            pltpu.sync_copy(rows_b, out_hbm.at[pl.ds(b1, _GC)])
            return carry

        lax.fori_loop(0, npair, body, 0)

    return gk(table, idx_il)


def _heads_body(fea_ref, wg1_ref, bg1_ref, w2c_ref, wm1_ref, bm1_ref,
                wm2_ref, bm2_ref, gates_ref, msgs_ref, gmax_ref, *, cdtype):
    fea = fea_ref[...].astype(cdtype)
    hgs, msgs = [], []
    for h in range(NHEADS):
        hg = jnp.dot(fea, wg1_ref[h].astype(cdtype),
                     preferred_element_type=jnp.float32) + bg1_ref[h]
        hg = hg * jax.nn.sigmoid(hg)
        hgs.append(hg.astype(cdtype))
        hm = jnp.dot(fea, wm1_ref[h].astype(cdtype),
                     preferred_element_type=jnp.float32) + bm1_ref[h]
        hm = hm * jax.nn.sigmoid(hm)
        m = jnp.dot(hm.astype(cdtype), wm2_ref[h].astype(cdtype),
                    preferred_element_type=jnp.float32) + bm2_ref[h]
        msgs.append(m)
    hg_cat = jnp.concatenate(hgs, axis=1)
    gates = jnp.dot(hg_cat, w2c_ref[...].astype(cdtype),
                    preferred_element_type=jnp.float32)
    gates_ref[...] = gates
    msgs_ref[...] = jnp.concatenate(msgs, axis=1).astype(msgs_ref.dtype)
    bmax = jnp.max(gates, axis=0, keepdims=True)

    @pl.when(pl.program_id(0) == 0)
    def _():
        gmax_ref[...] = bmax

    @pl.when(pl.program_id(0) > 0)
    def _():
        gmax_ref[...] = jnp.maximum(gmax_ref[...], bmax)


def _stack_heads(heads):
    wg1 = jnp.stack([h["gate"]["hidden"][0][0] for h in heads])
    bg1 = jnp.stack([h["gate"]["hidden"][0][1] for h in heads])
    hid = wg1.shape[-1]
    w2c = jnp.concatenate([
        jnp.pad(h["gate"]["out"][0], ((0, 0), (i, 8 - 1 - i)))
        for i, h in enumerate(heads)], axis=0)
    wm1 = jnp.stack([h["msg"]["hidden"][0][0] for h in heads])
    bm1 = jnp.stack([h["msg"]["hidden"][0][1] for h in heads])
    wm2 = jnp.stack([h["msg"]["out"][0] for h in heads])
    bm2 = jnp.stack([h["msg"]["out"][1] for h in heads])
    return (wg1, bg1, w2c, wm1, bm1, wm2, bm2), hid


def _heads_forward(fea, stacked, hid, be, cdtype):
    e, din = fea.shape
    wg1, bg1, w2c, wm1, bm1, wm2, bm2 = stacked
    full = lambda *shape: pl.BlockSpec(shape, lambda i: (0,) * len(shape))
    return pl.pallas_call(
        functools.partial(_heads_body, cdtype=cdtype),
        grid=(e // be,),
        in_specs=[
            pl.BlockSpec((be, din), lambda i: (i, 0)),
            full(NHEADS, din, hid), full(NHEADS, hid),
            full(NHEADS * hid, 8),
            full(NHEADS, din, hid), full(NHEADS, hid),
            full(NHEADS, hid, F), full(NHEADS, F),
        ],
        out_specs=[
            pl.BlockSpec((be, 8), lambda i: (i, 0)),
            pl.BlockSpec((be, NHEADS * F), lambda i: (i, 0)),
            pl.BlockSpec((1, 8), lambda i: (0, 0)),
        ],
        out_shape=[
            jax.ShapeDtypeStruct((e, 8), jnp.float32),
            jax.ShapeDtypeStruct((e, NHEADS * F), jnp.bfloat16),
            jax.ShapeDtypeStruct((1, 8), jnp.float32),
        ],
        compiler_params=pltpu.CompilerParams(
            dimension_semantics=("arbitrary",)),
    )(fea, wg1, bg1, w2c, wm1, bm1, wm2, bm2)


def _sscan_body(gates_ref, gmax_ref, w_ref, msgs_ref, idx_ref, out_ref,
                carry_ref, cidx_ref, *, sub, nsub):
    @pl.when(pl.program_id(0) == 0)
    def _():
        carry_ref[...] = jnp.zeros_like(carry_ref)
        cidx_ref[...] = jnp.full_like(cidx_ref, -1)

    p8 = w_ref[...] * jnp.exp(gates_ref[...] - gmax_ref[...])
    m = msgs_ref[...].astype(jnp.float32)
    parts = [p8[:, h:h + 1] * m[:, h * F:(h + 1) * F] for h in range(NHEADS)]
    parts.append(p8)
    val = jnp.concatenate(parts, axis=1)
    idx = idx_ref[...]
    ri = jax.lax.broadcasted_iota(jnp.int32, (sub, sub), 0)
    ci = jax.lax.broadcasted_iota(jnp.int32, (sub, sub), 1)
    tri = ci <= ri
    for g in range(nsub):
        idxg = idx[g * sub:(g + 1) * sub, :]
        valg = val[g * sub:(g + 1) * sub, :].astype(jnp.bfloat16)
        idr = jax.lax.broadcast_in_dim(idxg[:, 0], (sub, sub), (0,))
        idc = jax.lax.broadcast_in_dim(idxg[:, 0], (sub, sub), (1,))
        mask = jnp.logical_and(idr == idc, tri).astype(jnp.bfloat16)
        ssum = jnp.dot(mask, valg, preferred_element_type=jnp.float32)
        ssum = ssum + ((idxg == cidx_ref[...]).astype(jnp.float32)
                       * carry_ref[...])
        out_ref[g * sub:(g + 1) * sub, :] = ssum
        carry_ref[...] = ssum[sub - 1:sub, :]
        cidx_ref[...] = idxg[sub - 1:sub, :]


def _seg_scan(gates8, gmax, edge_w, msgs, idx2, be, sub):
    e = gates8.shape[0]
    return pl.pallas_call(
        functools.partial(_sscan_body, sub=sub, nsub=be // sub),
        grid=(e // be,),
        in_specs=[
            pl.BlockSpec((be, 8), lambda i: (i, 0)),
            pl.BlockSpec((1, 8), lambda i: (0, 0)),
            pl.BlockSpec((be, 1), lambda i: (i, 0)),
            pl.BlockSpec((be, NHEADS * F), lambda i: (i, 0)),
            pl.BlockSpec((be, 1), lambda i: (i, 0)),
        ],
        out_specs=pl.BlockSpec((be, SCW), lambda i: (i, 0)),
        out_shape=jax.ShapeDtypeStruct((e, SCW), jnp.float32),
        scratch_shapes=[
            pltpu.VMEM((1, SCW), jnp.float32),
            pltpu.VMEM((1, 1), jnp.int32),
        ],
        compiler_params=pltpu.CompilerParams(
            dimension_semantics=("arbitrary",)),
    )(gates8, gmax, edge_w, msgs, idx2)


def _wap(fea, idx, ends, counts, edge_w, stacked, hid, be, sub, cdtype):
    gates8, msgs, gmax = _heads_forward(fea, stacked, hid, be, cdtype)
    idx2 = idx.reshape(-1, 1)
    ssum = _seg_scan(gates8, gmax, edge_w, msgs, idx2, be, sub)
    pos = jnp.clip(ends - 1, 0, ssum.shape[0] - 1)
    s = jnp.where((counts > 0)[:, None], ssum[pos], 0.0)
    acc = s[:, 0:F] / (s[:, NHEADS * F:NHEADS * F + 1] + 1e-10)
    for h in range(1, NHEADS):
        acc = acc + (s[:, h * F:(h + 1) * F]
                     / (s[:, NHEADS * F + h:NHEADS * F + h + 1] + 1e-10))
    return acc * (1.0 / NHEADS)


def kernel(elem_weights, elem_fea, self_fea_idx, nbr_fea_idx, cry_elem_idx,
           params):
    cdtype = jnp.bfloat16
    n = elem_fea.shape[0]
    emb_W, emb_b = params["emb"]
    x = _embed(elem_fea, elem_weights, emb_W, emb_b)

    nbr_w = elem_weights[nbr_fea_idx]
    e = self_fea_idx.shape[0]
    idx_il = jnp.stack([self_fea_idx, nbr_fea_idx], axis=1).reshape(-1)
    counts = jnp.zeros((n,), jnp.int32).at[self_fea_idx].add(1)
    ends = jnp.cumsum(counts)
    for heads in params["graphs"]:
        stacked, hid = _stack_heads(heads)
        fea = _sc_gather(x, idx_il).reshape(e, 2 * F)
        pooled = _wap(fea, self_fea_idx, ends, counts, nbr_w, stacked, hid,
                      be=3200, sub=320, cdtype=cdtype)
        x = pooled + x

    cry_counts = jnp.zeros((2000,), jnp.int32).at[cry_elem_idx].add(1)
    cry_ends = jnp.cumsum(cry_counts)
    cry_stacked, cry_hid = _stack_heads(params["cry"])
    cry_fea = _wap(x, cry_elem_idx, cry_ends, cry_counts, elem_weights,
                   cry_stacked, cry_hid, be=1000, sub=250, cdtype=cdtype)
    return (cry_fea, x)
